# baseline (speedup 1.0000x reference)
import functools

import jax
import jax.numpy as jnp
from jax import lax
from jax.experimental import pallas as pl
from jax.experimental.pallas import tpu as pltpu
from jax.experimental.pallas import tpu_sc as plsc

_N = 10000
_E = 320000
_D = 128
_W = 144
_NT = 10016
_LAYERS = 4
_NC = 2
_NS = 16
_NW = _NC * _NS
_B = 64
_EPW = 10240
_EP = _NW * _EPW
_NBLK = _EPW // _B
_NP = 10240
_RPS = _NP // _NS


def _build_aug(x):
    ss = jnp.sum(x * x, axis=1, keepdims=True)
    invn = 1.0 / (jnp.sqrt(ss) + 1e-12)
    ones = jnp.ones((_N, 1), jnp.float32)
    zpad = jnp.zeros((_N, _W - _D - 2), jnp.float32)
    rows = jnp.concatenate([x, ones, invn, zpad], axis=1)
    return jnp.concatenate([rows, jnp.zeros((_NT - _N, _W), jnp.float32)], axis=0)


def _prep0_body(x_ref, aug_ref):
    aug_ref[...] = _build_aug(x_ref[...])


def _prep0(x):
    return pl.pallas_call(
        _prep0_body,
        out_shape=jax.ShapeDtypeStruct((_NT, _W), jnp.float32),
    )(x)


def _layer_x(acc_ref):
    a = acc_ref[0, :_N] + acc_ref[1, :_N]
    den = a[:, _D]
    return jnp.maximum(a[:, :_D] / (den[:, None] + 1e-12), 0.0)


def _combine_mid_body(acc_ref, aug_ref):
    aug_ref[...] = _build_aug(_layer_x(acc_ref))


def _combine_mid(acc):
    return pl.pallas_call(
        _combine_mid_body,
        out_shape=jax.ShapeDtypeStruct((_NT, _W), jnp.float32),
    )(acc)


def _combine_final_body(acc_ref, x_ref):
    x_ref[...] = _layer_x(acc_ref)


def _combine_final(acc):
    return pl.pallas_call(
        _combine_final_body,
        out_shape=jax.ShapeDtypeStruct((_N, _D), jnp.float32),
    )(acc)


_mesh = plsc.VectorSubcoreMesh(core_axis_name="c", subcore_axis_name="s")


@functools.partial(
    pl.kernel,
    out_type=jax.ShapeDtypeStruct((_NC, _NP, _W), jnp.float32),
    mesh=_mesh,
    compiler_params=pltpu.CompilerParams(
        needs_layout_passes=False, use_tc_tiling_on_sc=False),
    scratch_types=[
        pltpu.VMEM((4, _B), jnp.int32),
        pltpu.VMEM((4, _B), jnp.int32),
        pltpu.VMEM((_B, _W), jnp.float32),
        pltpu.VMEM((_B, _W), jnp.float32),
        pltpu.VMEM((_B, _W), jnp.float32),
        pltpu.VMEM((_B, _W), jnp.float32),
        pltpu.VMEM((16, 16), jnp.float32),
        pltpu.VMEM((16,), jnp.float32),
        pltpu.VMEM_SHARED((_NP, _W), jnp.float32),
        pltpu.SemaphoreType.DMA,
        pltpu.SemaphoreType.DMA,
        pltpu.SemaphoreType.DMA,
        pltpu.SemaphoreType.DMA,
        pltpu.SemaphoreType.DMA,
        pltpu.SemaphoreType.DMA,
        pltpu.SemaphoreType.DMA,
        pltpu.SemaphoreType.DMA,
    ],
)
def _edge(aug_hbm, src_hbm, dst_hbm, beta_hbm, zrow_hbm, out_hbm,
          idx_s, idx_d, rows_s0, rows_s1, rows_d0, rows_d1, part, beta_v,
          acc_sh, sem_i0, sem_i1, sem_i2, sem_i3, sem_g0, sem_g1,
          sem_sc0, sem_sc1):
    cid = lax.axis_index("c")
    sid = lax.axis_index("s")
    wid = cid * _NS + sid
    base = wid * _EPW

    rows_s = (rows_s0, rows_s1)
    rows_d = (rows_d0, rows_d1)
    sem_i = (sem_i0, sem_i1, sem_i2, sem_i3)
    sem_g = (sem_g0, sem_g1)
    sem_sc = (sem_sc0, sem_sc1)

    pltpu.sync_copy(beta_hbm, beta_v)
    pltpu.sync_copy(zrow_hbm, acc_sh.at[pl.ds(sid * _RPS, _RPS)])
    plsc.subcore_barrier()

    lane = lax.iota(jnp.int32, 16)
    beta = beta_v[...]

    def issue_idx(b, r):
        off = base + b * _B
        pltpu.async_copy(src_hbm.at[pl.ds(off, _B)], idx_s.at[r], sem_i[r])
        pltpu.async_copy(dst_hbm.at[pl.ds(off, _B)], idx_d.at[r], sem_i[r])

    def wait_idx(r):
        pltpu.make_async_copy(src_hbm.at[pl.ds(0, _B)], idx_s.at[r], sem_i[r]).wait()
        pltpu.make_async_copy(src_hbm.at[pl.ds(0, _B)], idx_d.at[r], sem_i[r]).wait()

    def issue_gather(r, p):
        pltpu.async_copy(aug_hbm.at[idx_s.at[r]], rows_s[p], sem_g[p])
        pltpu.async_copy(aug_hbm.at[idx_d.at[r]], rows_d[p], sem_g[p])

    def wait_gather(p):
        pltpu.make_async_copy(zrow_hbm.at[pl.ds(0, _B)], rows_s[p], sem_g[p]).wait()
        pltpu.make_async_copy(zrow_hbm.at[pl.ds(0, _B)], rows_d[p], sem_g[p]).wait()

    def issue_scatter(r, p):
        pltpu.async_copy(rows_s[p], acc_sh.at[idx_d.at[r]], sem_sc[p],
                         add=True)

    def wait_scatter(p):
        pltpu.make_async_copy(zrow_hbm.at[pl.ds(0, _B)], rows_s[p], sem_sc[p]).wait()

    def compute(p):
        rs = rows_s[p]
        rd = rows_d[p]

        def group(g, c2):
            gb = g * 16
            col = jnp.full((16,), _D + 1, jnp.int32)
            inv_s = plsc.load_gather(rs, [gb + lane, col])
            inv_d = plsc.load_gather(rd, [gb + lane, col])
            for q in range(16):
                e = gb + q
                prod = rs[e, pl.ds(0, 16)] * rd[e, pl.ds(0, 16)]
                for j in range(1, 8):
                    prod = prod + rs[e, pl.ds(16 * j, 16)] * rd[e, pl.ds(16 * j, 16)]
                part[q, :] = prod
            dots = plsc.load_gather(part, [lane, jnp.zeros((16,), jnp.int32)])
            for c in range(1, 16):
                dots = dots + plsc.load_gather(part, [lane, jnp.full((16,), c, jnp.int32)])
            cos = dots * inv_s * inv_d
            ex = jnp.exp(beta * cos)
            for q in range(16):
                e = gb + q
                coef = ex[q]
                for j in range(9):
                    rs[e, pl.ds(16 * j, 16)] = rs[e, pl.ds(16 * j, 16)] * coef
            return c2

        lax.fori_loop(0, _B // 16, group, 0)

    issue_idx(0, 0)
    issue_idx(1, 1)
    wait_idx(0)
    issue_gather(0, 0)

    def outer(i, carry):
        for u in range(4):
            b = 4 * i + u
            p = u % 2
            q = 1 - p
            r = u
            rn = (u + 1) % 4
            ri = (u + 2) % 4

            @pl.when(b >= 1)
            def _():
                wait_scatter(q)

            @pl.when(b + 1 < _NBLK)
            def _():
                wait_idx(rn)
                issue_gather(rn, q)

            @pl.when(b + 2 < _NBLK)
            def _():
                issue_idx(b + 2, ri)

            wait_gather(p)
            compute(p)
            issue_scatter(r, p)
        return carry

    lax.fori_loop(0, _NBLK // 4, outer, 0)

    wait_scatter(1)
    plsc.subcore_barrier()
    pltpu.sync_copy(acc_sh.at[pl.ds(sid * _RPS, _RPS)],
                    out_hbm.at[cid, pl.ds(sid * _RPS, _RPS)])


def kernel(feats, edge_index, betas):
    pad = jnp.full((_EP - _E,), _N, jnp.int32)
    src = jnp.concatenate([edge_index[0], pad])
    dst = jnp.concatenate([edge_index[1], pad])
    zrow = jnp.zeros((_RPS, _W), jnp.float32)
    aug = _prep0(feats)
    x = feats
    for i in range(_LAYERS):
        beta_vec = jnp.full((16,), betas[i], jnp.float32)
        acc = _edge(aug, src, dst, beta_vec, zrow)
        if i < _LAYERS - 1:
            aug = _combine_mid(acc)
        else:
            x = _combine_final(acc)
    return x
